# baseline (speedup 1.0000x reference)
import functools

import jax
import jax.numpy as jnp
from jax.experimental import pallas as pl
from jax.experimental.pallas import tpu as pltpu

_TMP = 0.1
_FMP = 0.1


def _mask_body(t0_ref, f0_ref, x_ref, o_ref, *, tlen, flen):
    b = pl.program_id(0)
    t0 = t0_ref[b]
    f0 = f0_ref[b]
    x = x_ref[0]
    fdim, tdim = x.shape
    cols = jax.lax.broadcasted_iota(jnp.int32, (fdim, tdim), 1)
    rows = jax.lax.broadcasted_iota(jnp.int32, (fdim, tdim), 0)
    keep = ((cols < t0) | (cols >= t0 + tlen)) & (
        (rows < f0) | (rows >= f0 + flen)
    )
    o_ref[0] = jnp.where(keep, x, jnp.float32(0.0))


def kernel(spec):
    B, C, Fd, T = spec.shape
    tlen = int(T * _TMP)
    flen = int(Fd * _FMP)
    kt = jax.random.fold_in(jax.random.key(1), 0)
    t0 = jax.random.randint(kt, (B,), 0, max(1, T - tlen + 1))
    kf = jax.random.fold_in(jax.random.key(1), 1)
    f0 = jax.random.randint(kf, (B,), 0, max(1, Fd - flen + 1))

    x = spec.reshape(B, C * Fd, T)
    grid_spec = pltpu.PrefetchScalarGridSpec(
        num_scalar_prefetch=2,
        grid=(B,),
        in_specs=[pl.BlockSpec((1, C * Fd, T), lambda b, t0_ref, f0_ref: (b, 0, 0))],
        out_specs=pl.BlockSpec((1, C * Fd, T), lambda b, t0_ref, f0_ref: (b, 0, 0)),
    )
    out = pl.pallas_call(
        functools.partial(_mask_body, tlen=tlen, flen=flen),
        grid_spec=grid_spec,
        out_shape=jax.ShapeDtypeStruct(x.shape, x.dtype),
    )(t0, f0, x)
    return out.reshape(B, C, Fd, T)
